# baseline (speedup 1.0000x reference)
import jax
import jax.numpy as jnp
from jax.experimental import pallas as pl

_NUM_CLASSES = 81
_BLK = 4000


def _onehot_body(lab_ref, out_ref):
    l = lab_ref[...]
    pos = jnp.abs(l)
    sign = jnp.where(l < 0, -1, 1)
    iota = jax.lax.broadcasted_iota(jnp.int32, (_BLK, _NUM_CLASSES), 1)
    out_ref[...] = jnp.where(iota == pos, sign, 0)


def kernel(cls_label):
    n = cls_label.size
    labels = jnp.reshape(cls_label, (n, 1))
    grid = n // _BLK
    out = pl.pallas_call(
        _onehot_body,
        grid=(grid,),
        in_specs=[pl.BlockSpec((_BLK, 1), lambda i: (i, 0))],
        out_specs=pl.BlockSpec((_BLK, _NUM_CLASSES), lambda i: (i, 0)),
        out_shape=jax.ShapeDtypeStruct((n, _NUM_CLASSES), jnp.int32),
    )(labels)
    return jnp.reshape(out, cls_label.shape + (_NUM_CLASSES,))


# speedup vs baseline: 1.0511x; 1.0511x over previous
import jax
import jax.numpy as jnp
from jax.experimental import pallas as pl

_NUM_CLASSES = 81
_BLK = 4000


def _onehot_body(lab_ref, out_ref):
    lf = lab_ref[...].astype(jnp.float32)
    ones = jnp.ones((1, _NUM_CLASSES), jnp.float32)
    lb = jax.lax.dot_general(
        lf, ones, (((1,), (0,)), ((), ())),
        preferred_element_type=jnp.float32)
    iota = jax.lax.broadcasted_iota(
        jnp.int32, (_BLK, _NUM_CLASSES), 1).astype(jnp.float32)
    out_ref[...] = jnp.where(lb == iota, 1,
                             jnp.where(lb == -iota, -1, 0)).astype(jnp.int32)


def kernel(cls_label):
    n = cls_label.size
    labels = jnp.reshape(cls_label, (n, 1))
    grid = n // _BLK
    out = pl.pallas_call(
        _onehot_body,
        grid=(grid,),
        in_specs=[pl.BlockSpec((_BLK, 1), lambda i: (i, 0))],
        out_specs=pl.BlockSpec((_BLK, _NUM_CLASSES), lambda i: (i, 0)),
        out_shape=jax.ShapeDtypeStruct((n, _NUM_CLASSES), jnp.int32),
    )(labels)
    return jnp.reshape(out, cls_label.shape + (_NUM_CLASSES,))


# speedup vs baseline: 2.5147x; 2.3925x over previous
import jax
import jax.numpy as jnp
from jax.experimental import pallas as pl

_NUM_CLASSES = 81
_BLK = 4000


def _onehot_body(lab_ref, out_ref):
    lf = lab_ref[0].astype(jnp.float32)
    ones = jnp.ones((1, _NUM_CLASSES), jnp.float32)
    lb = jax.lax.dot_general(
        lf, ones, (((0,), (0,)), ((), ())),
        preferred_element_type=jnp.float32)
    iota = jax.lax.broadcasted_iota(
        jnp.int32, (_BLK, _NUM_CLASSES), 1).astype(jnp.float32)
    out_ref[...] = jnp.where(lb == iota, 1,
                             jnp.where(lb == -iota, -1, 0)).astype(jnp.int32)


def kernel(cls_label):
    n = cls_label.size
    grid = n // _BLK
    labels = jnp.reshape(cls_label, (grid, 1, _BLK))
    out = pl.pallas_call(
        _onehot_body,
        grid=(grid,),
        in_specs=[pl.BlockSpec((1, 1, _BLK), lambda i: (i, 0, 0))],
        out_specs=pl.BlockSpec((_BLK, _NUM_CLASSES), lambda i: (i, 0)),
        out_shape=jax.ShapeDtypeStruct((n, _NUM_CLASSES), jnp.int32),
    )(labels)
    return jnp.reshape(out, cls_label.shape + (_NUM_CLASSES,))
